# Initial kernel scaffold; baseline (speedup 1.0000x reference)
import jax
import jax.numpy as jnp
from jax.experimental import pallas as pl


def kernel(x, edge_index, W1, b1, W2, b2):
    raise NotImplementedError("write your pallas kernel here")


# speedup vs baseline: 4.0920x; 4.0920x over previous
import functools

import jax
import jax.numpy as jnp
from jax import lax
from jax.experimental import pallas as pl
from jax.experimental.pallas import tpu as pltpu
from jax.experimental.pallas import tpu_sc as plsc

N = 10000
E = 320000
D = 128

NUM_CORES = 2
NUM_SUBCORES = 16
NW = NUM_CORES * NUM_SUBCORES
EDGES_PER_W = E // NW
CHUNK = 128
EP = 10240
NCHUNK = EP // CHUNK
NP = 10240
ROWS_PER_TILE = NP // NUM_SUBCORES
NCOPY = ROWS_PER_TILE // CHUNK


def _sc_segment_body(x_hbm, srcg_hbm, dstg_hbm,
                     m_parts_hbm, deg_parts_hbm,
                     m_sh, src_v, dst_v, rows_v, deg_v, sem):
  c = lax.axis_index("c")
  s = lax.axis_index("s")
  wid = c * NUM_SUBCORES + s

  z16 = jnp.zeros((16,), jnp.float32)

  def zrow(j, _):
    for k in range(D // 16):
      rows_v[j, pl.ds(k * 16, 16)] = z16
    return 0
  lax.fori_loop(0, CHUNK, zrow, 0)

  my_base = s * ROWS_PER_TILE
  for t in range(NCOPY):
    pltpu.sync_copy(rows_v, m_sh.at[pl.ds(my_base + t * CHUNK, CHUNK)])

  def zdeg(i, _):
    deg_v[pl.ds(i * 16, 16)] = z16
    return 0
  lax.fori_loop(0, NP // 16, zdeg, 0)

  pltpu.sync_copy(srcg_hbm.at[wid], src_v)
  pltpu.sync_copy(dstg_hbm.at[wid], dst_v)

  plsc.subcore_barrier()

  def chunk_body(j, _):
    pltpu.async_copy(x_hbm.at[src_v.at[j]], rows_v, sem).wait()
    pltpu.sync_copy(rows_v, m_sh.at[dst_v.at[j]], add=True)
    return 0
  lax.fori_loop(0, NCHUNK, chunk_body, 0)

  ones16 = jnp.ones((16,), jnp.float32)
  def deg_body(j, _):
    for k in range(CHUNK // 16):
      idx = dst_v[j, pl.ds(k * 16, 16)]
      plsc.addupdate_scatter(deg_v, [idx], ones16)
    return 0
  lax.fori_loop(0, NCHUNK, deg_body, 0)
  pltpu.sync_copy(deg_v, deg_parts_hbm.at[pl.ds(wid * NP, NP)])

  plsc.subcore_barrier()

  for t in range(NCOPY):
    base = my_base + t * CHUNK
    pltpu.sync_copy(m_sh.at[pl.ds(base, CHUNK)], rows_v)
    pltpu.sync_copy(rows_v, m_parts_hbm.at[c, pl.ds(base, CHUNK)])


def _sc_segment_sum(x, srcg, dstg):
  mesh = plsc.VectorSubcoreMesh(core_axis_name="c", subcore_axis_name="s")
  fn = functools.partial(
      pl.kernel,
      mesh=mesh,
      compiler_params=pltpu.CompilerParams(
          needs_layout_passes=False, use_tc_tiling_on_sc=False),
      out_type=[
          jax.ShapeDtypeStruct((NUM_CORES, NP, D), jnp.float32),
          jax.ShapeDtypeStruct((NW * NP,), jnp.float32),
      ],
      scratch_types=[
          pltpu.VMEM_SHARED((NP, D), jnp.float32),
          pltpu.VMEM((NCHUNK, CHUNK), jnp.int32),
          pltpu.VMEM((NCHUNK, CHUNK), jnp.int32),
          pltpu.VMEM((CHUNK, D), jnp.float32),
          pltpu.VMEM((NP,), jnp.float32),
          pltpu.SemaphoreType.DMA,
      ],
  )(_sc_segment_body)
  return fn(x, srcg, dstg)


BLK = 1000


def _tc_body(x_ref, mp_ref, degp_ref, w1_ref, b1_ref, w2_ref, b2_ref, o_ref):
  dn = (((1,), (1,)), ((), ()))
  xn = lax.dot_general(x_ref[...], w1_ref[...], dn,
                       preferred_element_type=jnp.float32) + b1_ref[...]
  m0 = mp_ref[0] + mp_ref[1]
  mm = lax.dot_general(m0, w2_ref[...], dn,
                       preferred_element_type=jnp.float32) + b2_ref[...]
  deg = jnp.sum(degp_ref[...], axis=1)[:, None]
  o_ref[...] = xn + mm / deg


def _tc_combine(x, m_parts, deg_parts, W1, b1, W2, b2):
  grid = (N // BLK,)
  return pl.pallas_call(
      _tc_body,
      grid=grid,
      in_specs=[
          pl.BlockSpec((BLK, D), lambda i: (i, 0)),
          pl.BlockSpec((NUM_CORES, BLK, D), lambda i: (0, i, 0)),
          pl.BlockSpec((BLK, NW), lambda i: (i, 0)),
          pl.BlockSpec((D, D), lambda i: (0, 0)),
          pl.BlockSpec((1, D), lambda i: (0, 0)),
          pl.BlockSpec((D, D), lambda i: (0, 0)),
          pl.BlockSpec((1, D), lambda i: (0, 0)),
      ],
      out_specs=pl.BlockSpec((BLK, D), lambda i: (i, 0)),
      out_shape=jax.ShapeDtypeStruct((N, D), jnp.float32),
  )(x, m_parts, deg_parts, W1, b1, W2, b2)


@jax.jit
def kernel(x, edge_index, W1, b1, W2, b2):
  dst = edge_index[0]
  src = edge_index[1]
  pad = EP - EDGES_PER_W
  srcg = jnp.pad(src.reshape(NW, EDGES_PER_W), ((0, 0), (0, pad)),
                 constant_values=0).reshape(NW, NCHUNK, CHUNK)
  dstg = jnp.pad(dst.reshape(NW, EDGES_PER_W), ((0, 0), (0, pad)),
                 constant_values=N).reshape(NW, NCHUNK, CHUNK)
  m_parts, deg_parts = _sc_segment_sum(x, srcg, dstg)
  deg_t = deg_parts.reshape(NW, NP)[:, :N].T
  return _tc_combine(x, m_parts, deg_t,
                     W1, b1.reshape(1, D), W2, b2.reshape(1, D))


# speedup vs baseline: 4.5363x; 1.1086x over previous
import functools

import jax
import jax.numpy as jnp
from jax import lax
from jax.experimental import pallas as pl
from jax.experimental.pallas import tpu as pltpu
from jax.experimental.pallas import tpu_sc as plsc

N = 10000
E = 320000
D = 128

NUM_CORES = 2
NUM_SUBCORES = 16
NW = NUM_CORES * NUM_SUBCORES
EDGES_PER_W = E // NW
CHUNK = 128
EP = 10240
NCHUNK = EP // CHUNK
NP = 10240
ROWS_PER_TILE = NP // NUM_SUBCORES
NCOPY = ROWS_PER_TILE // CHUNK


def _sc_segment_body(x_hbm, idx2_hbm,
                     m_parts_hbm, deg_parts_hbm,
                     m_sh, rows0_v, rows1_v, idxc_v, deg_v,
                     gsem0, gsem1, ssem0, ssem1):
  c = lax.axis_index("c")
  s = lax.axis_index("s")
  wid = c * NUM_SUBCORES + s
  rows = (rows0_v, rows1_v)
  gsem = (gsem0, gsem1)
  ssem = (ssem0, ssem1)

  z16 = jnp.zeros((16,), jnp.float32)
  ones16 = jnp.ones((16,), jnp.float32)

  def zrow(j, _):
    for k in range(D // 16):
      rows0_v[j, pl.ds(k * 16, 16)] = z16
    return 0
  lax.fori_loop(0, CHUNK, zrow, 0)

  my_base = s * ROWS_PER_TILE
  for t in range(NCOPY):
    pltpu.sync_copy(rows0_v, m_sh.at[pl.ds(my_base + t * CHUNK, CHUNK)])

  def zdeg(i, _):
    deg_v[pl.ds(i * 16, 16)] = z16
    return 0
  lax.fori_loop(0, NP // 16, zdeg, 0)

  plsc.subcore_barrier()

  def load_idx(b, j):
    pltpu.sync_copy(idx2_hbm.at[wid, j], idxc_v.at[b])

  def gather_start(b):
    pltpu.async_copy(x_hbm.at[idxc_v.at[b, 0]], rows[b], gsem[b])

  def gather_wait(b):
    pltpu.make_async_copy(x_hbm.at[idxc_v.at[b, 0]], rows[b], gsem[b]).wait()

  def scatter_start(b):
    return pltpu.async_copy(rows[b], m_sh.at[idxc_v.at[b, 1]], ssem[b],
                            add=True)

  def histogram(b):
    for k in range(CHUNK // 16):
      idx = idxc_v[b, 1, pl.ds(k * 16, 16)]
      plsc.addupdate_scatter(deg_v, [idx], ones16)

  load_idx(0, 0)
  gather_start(0)
  load_idx(1, 1)
  gather_start(1)

  def pipe_body(g, _):
    for b in range(2):
      gather_wait(b)
      sc_desc = scatter_start(b)
      histogram(b)
      sc_desc.wait()
      load_idx(b, 2 * g + b + 2)
      gather_start(b)
    return 0
  lax.fori_loop(0, NCHUNK // 2 - 1, pipe_body, 0)

  for b in range(2):
    gather_wait(b)
    sc_desc = scatter_start(b)
    histogram(b)
    sc_desc.wait()

  pltpu.sync_copy(deg_v, deg_parts_hbm.at[pl.ds(wid * NP, NP)])

  plsc.subcore_barrier()

  for t in range(NCOPY):
    base = my_base + t * CHUNK
    pltpu.sync_copy(m_sh.at[pl.ds(base, CHUNK)], rows0_v)
    pltpu.sync_copy(rows0_v, m_parts_hbm.at[c, pl.ds(base, CHUNK)])


def _sc_segment_sum(x, idx2):
  mesh = plsc.VectorSubcoreMesh(core_axis_name="c", subcore_axis_name="s")
  fn = functools.partial(
      pl.kernel,
      mesh=mesh,
      compiler_params=pltpu.CompilerParams(
          needs_layout_passes=False, use_tc_tiling_on_sc=False),
      out_type=[
          jax.ShapeDtypeStruct((NUM_CORES, NP, D), jnp.float32),
          jax.ShapeDtypeStruct((NW * NP,), jnp.float32),
      ],
      scratch_types=[
          pltpu.VMEM_SHARED((NP, D), jnp.float32),
          pltpu.VMEM((CHUNK, D), jnp.float32),
          pltpu.VMEM((CHUNK, D), jnp.float32),
          pltpu.VMEM((2, 2, CHUNK), jnp.int32),
          pltpu.VMEM((NP,), jnp.float32),
          pltpu.SemaphoreType.DMA,
          pltpu.SemaphoreType.DMA,
          pltpu.SemaphoreType.DMA,
          pltpu.SemaphoreType.DMA,
      ],
  )(_sc_segment_body)
  return fn(x, idx2)


BLK = 1000


def _tc_body(x_ref, mp_ref, degp_ref, w1_ref, b1_ref, w2_ref, b2_ref, o_ref):
  dn = (((1,), (1,)), ((), ()))
  xn = lax.dot_general(x_ref[...], w1_ref[...], dn,
                       preferred_element_type=jnp.float32) + b1_ref[...]
  m0 = mp_ref[0] + mp_ref[1]
  mm = lax.dot_general(m0, w2_ref[...], dn,
                       preferred_element_type=jnp.float32) + b2_ref[...]
  deg = jnp.sum(degp_ref[...], axis=1)[:, None]
  o_ref[...] = xn + mm / deg


def _tc_combine(x, m_parts, deg_parts, W1, b1, W2, b2):
  grid = (N // BLK,)
  return pl.pallas_call(
      _tc_body,
      grid=grid,
      in_specs=[
          pl.BlockSpec((BLK, D), lambda i: (i, 0)),
          pl.BlockSpec((NUM_CORES, BLK, D), lambda i: (0, i, 0)),
          pl.BlockSpec((BLK, NW), lambda i: (i, 0)),
          pl.BlockSpec((D, D), lambda i: (0, 0)),
          pl.BlockSpec((1, D), lambda i: (0, 0)),
          pl.BlockSpec((D, D), lambda i: (0, 0)),
          pl.BlockSpec((1, D), lambda i: (0, 0)),
      ],
      out_specs=pl.BlockSpec((BLK, D), lambda i: (i, 0)),
      out_shape=jax.ShapeDtypeStruct((N, D), jnp.float32),
  )(x, m_parts, deg_parts, W1, b1, W2, b2)


@jax.jit
def kernel(x, edge_index, W1, b1, W2, b2):
  dst = edge_index[0]
  src = edge_index[1]
  pad = EP - EDGES_PER_W
  srcg = jnp.pad(src.reshape(NW, EDGES_PER_W), ((0, 0), (0, pad)),
                 constant_values=0).reshape(NW, NCHUNK, CHUNK)
  dstg = jnp.pad(dst.reshape(NW, EDGES_PER_W), ((0, 0), (0, pad)),
                 constant_values=N).reshape(NW, NCHUNK, CHUNK)
  idx2 = jnp.stack([srcg, dstg], axis=2)
  m_parts, deg_parts = _sc_segment_sum(x, idx2)
  deg_t = deg_parts.reshape(NW, NP)[:, :N].T
  return _tc_combine(x, m_parts, deg_t,
                     W1, b1.reshape(1, D), W2, b2.reshape(1, D))


# speedup vs baseline: 4.6448x; 1.0239x over previous
import functools

import jax
import jax.numpy as jnp
from jax import lax
from jax.experimental import pallas as pl
from jax.experimental.pallas import tpu as pltpu
from jax.experimental.pallas import tpu_sc as plsc

N = 10000
E = 320000
D = 128

NUM_CORES = 2
NUM_SUBCORES = 16
NW = NUM_CORES * NUM_SUBCORES
EDGES_PER_W = E // NW
CHUNK = 64
EP = 10240
NCHUNK = EP // CHUNK
NBUF = 4
BLK_CH = 16
NBLK = NCHUNK // BLK_CH
NP = 10240
ROWS_PER_TILE = NP // NUM_SUBCORES
COPY_CH = 64
NCOPY = ROWS_PER_TILE // COPY_CH


def _sc_segment_body(x_hbm, idx2_hbm,
                     m_parts_hbm, deg_parts_hbm,
                     m_sh, rows0_v, rows1_v, rows2_v, rows3_v,
                     idxb_v, deg_v,
                     gsem0, gsem1, gsem2, gsem3, ssem, isem):
  c = lax.axis_index("c")
  s = lax.axis_index("s")
  wid = c * NUM_SUBCORES + s
  rows = (rows0_v, rows1_v, rows2_v, rows3_v)
  gsem = (gsem0, gsem1, gsem2, gsem3)

  z16 = jnp.zeros((16,), jnp.float32)
  ones16 = jnp.ones((16,), jnp.float32)

  def zrow(j, _):
    for k in range(D // 16):
      rows0_v[j, pl.ds(k * 16, 16)] = z16
    return 0
  lax.fori_loop(0, COPY_CH, zrow, 0)

  my_base = s * ROWS_PER_TILE
  for t in range(NCOPY):
    pltpu.sync_copy(rows0_v, m_sh.at[pl.ds(my_base + t * COPY_CH, COPY_CH)])

  def zdeg(i, _):
    deg_v[pl.ds(i * 16, 16)] = z16
    return 0
  lax.fori_loop(0, NP // 16, zdeg, 0)

  plsc.subcore_barrier()

  def slot(j):
    return (j // BLK_CH) % 2, j % BLK_CH

  def idx_load_start(blk):
    return pltpu.async_copy(idx2_hbm.at[wid, pl.ds(blk * BLK_CH, BLK_CH)],
                            idxb_v.at[blk % 2], isem)

  def gather_start(j):
    b = j % NBUF
    p, r = slot(j)
    return pltpu.async_copy(x_hbm.at[idxb_v.at[p, r, 0]], rows[b], gsem[b])

  def scatter_start(j):
    b = j % NBUF
    p, r = slot(j)
    return pltpu.async_copy(rows[b], m_sh.at[idxb_v.at[p, r, 1]], ssem,
                            add=True)

  def histogram(j):
    p, r = slot(j)
    for k in range(CHUNK // 16):
      idx = idxb_v[p, r, 1, pl.ds(k * 16, 16)]
      plsc.addupdate_scatter(deg_v, [idx], ones16)

  idx_load_start(0).wait()
  ipf = idx_load_start(1)
  ipf_waited = False
  gd = [None] * NBUF
  for j in range(NBUF):
    gd[j] = gather_start(j)

  for j in range(NCHUNK):
    ci = j % BLK_CH
    blk = j // BLK_CH
    if ci == 0 and blk >= 1:
      ipf = idx_load_start(blk + 1) if blk + 1 < NBLK else None
      ipf_waited = False
    if ci == BLK_CH - NBUF and ipf is not None and not ipf_waited:
      ipf.wait()
      ipf_waited = True
    b = j % NBUF
    gd[b].wait()
    sd = scatter_start(j)
    histogram(j)
    sd.wait()
    if j + NBUF < NCHUNK:
      gd[b] = gather_start(j + NBUF)

  pltpu.sync_copy(deg_v, deg_parts_hbm.at[pl.ds(wid * NP, NP)])

  plsc.subcore_barrier()

  for t in range(NCOPY):
    base = my_base + t * COPY_CH
    pltpu.sync_copy(m_sh.at[pl.ds(base, COPY_CH)], rows0_v)
    pltpu.sync_copy(rows0_v, m_parts_hbm.at[c, pl.ds(base, COPY_CH)])


def _sc_segment_sum(x, idx2):
  mesh = plsc.VectorSubcoreMesh(core_axis_name="c", subcore_axis_name="s")
  fn = functools.partial(
      pl.kernel,
      mesh=mesh,
      compiler_params=pltpu.CompilerParams(
          needs_layout_passes=False, use_tc_tiling_on_sc=False),
      out_type=[
          jax.ShapeDtypeStruct((NUM_CORES, NP, D), jnp.float32),
          jax.ShapeDtypeStruct((NW * NP,), jnp.float32),
      ],
      scratch_types=[
          pltpu.VMEM_SHARED((NP, D), jnp.float32),
          pltpu.VMEM((CHUNK, D), jnp.float32),
          pltpu.VMEM((CHUNK, D), jnp.float32),
          pltpu.VMEM((CHUNK, D), jnp.float32),
          pltpu.VMEM((CHUNK, D), jnp.float32),
          pltpu.VMEM((2, BLK_CH, 2, CHUNK), jnp.int32),
          pltpu.VMEM((NP,), jnp.float32),
          pltpu.SemaphoreType.DMA,
          pltpu.SemaphoreType.DMA,
          pltpu.SemaphoreType.DMA,
          pltpu.SemaphoreType.DMA,
          pltpu.SemaphoreType.DMA,
          pltpu.SemaphoreType.DMA,
      ],
  )(_sc_segment_body)
  return fn(x, idx2)


BLK = 1000


def _tc_body(x_ref, mp_ref, degp_ref, w1_ref, b1_ref, w2_ref, b2_ref, o_ref):
  dn = (((1,), (1,)), ((), ()))
  xn = lax.dot_general(x_ref[...], w1_ref[...], dn,
                       preferred_element_type=jnp.float32) + b1_ref[...]
  m0 = mp_ref[0] + mp_ref[1]
  mm = lax.dot_general(m0, w2_ref[...], dn,
                       preferred_element_type=jnp.float32) + b2_ref[...]
  deg = jnp.sum(degp_ref[...], axis=1)[:, None]
  o_ref[...] = xn + mm / deg


def _tc_combine(x, m_parts, deg_parts, W1, b1, W2, b2):
  grid = (N // BLK,)
  return pl.pallas_call(
      _tc_body,
      grid=grid,
      in_specs=[
          pl.BlockSpec((BLK, D), lambda i: (i, 0)),
          pl.BlockSpec((NUM_CORES, BLK, D), lambda i: (0, i, 0)),
          pl.BlockSpec((BLK, NW), lambda i: (i, 0)),
          pl.BlockSpec((D, D), lambda i: (0, 0)),
          pl.BlockSpec((1, D), lambda i: (0, 0)),
          pl.BlockSpec((D, D), lambda i: (0, 0)),
          pl.BlockSpec((1, D), lambda i: (0, 0)),
      ],
      out_specs=pl.BlockSpec((BLK, D), lambda i: (i, 0)),
      out_shape=jax.ShapeDtypeStruct((N, D), jnp.float32),
  )(x, m_parts, deg_parts, W1, b1, W2, b2)


@jax.jit
def kernel(x, edge_index, W1, b1, W2, b2):
  dst = edge_index[0]
  src = edge_index[1]
  pad = EP - EDGES_PER_W
  srcg = jnp.pad(src.reshape(NW, EDGES_PER_W), ((0, 0), (0, pad)),
                 constant_values=0).reshape(NW, NCHUNK, CHUNK)
  dstg = jnp.pad(dst.reshape(NW, EDGES_PER_W), ((0, 0), (0, pad)),
                 constant_values=N).reshape(NW, NCHUNK, CHUNK)
  idx2 = jnp.stack([srcg, dstg], axis=2)
  m_parts, deg_parts = _sc_segment_sum(x, idx2)
  deg_t = deg_parts.reshape(NW, NP)[:, :N].T
  return _tc_combine(x, m_parts, deg_t,
                     W1, b1.reshape(1, D), W2, b2.reshape(1, D))


# speedup vs baseline: 4.7035x; 1.0126x over previous
import functools

import jax
import jax.numpy as jnp
from jax import lax
from jax.experimental import pallas as pl
from jax.experimental.pallas import tpu as pltpu
from jax.experimental.pallas import tpu_sc as plsc

N = 10000
E = 320000
D = 128

NUM_CORES = 2
NUM_SUBCORES = 16
NW = NUM_CORES * NUM_SUBCORES
EDGES_PER_W = E // NW
CHUNK = 64
EP = 10240
NCHUNK = EP // CHUNK
NBUF = 4
BLK_CH = 16
NBLK = NCHUNK // BLK_CH
NP = 10240
ROWS_PER_TILE = NP // NUM_SUBCORES
COPY_CH = 64
NCOPY = ROWS_PER_TILE // COPY_CH


def _sc_segment_body(x_hbm, idx2_hbm,
                     m_parts_hbm, deg_parts_hbm,
                     m_sh, rows0_v, rows1_v, rows2_v, rows3_v,
                     idxb_v, deg_v,
                     gsem0, gsem1, gsem2, gsem3, ssem, isem):
  c = lax.axis_index("c")
  s = lax.axis_index("s")
  wid = c * NUM_SUBCORES + s
  rows = (rows0_v, rows1_v, rows2_v, rows3_v)
  gsem = (gsem0, gsem1, gsem2, gsem3)

  z16 = jnp.zeros((16,), jnp.float32)
  ones16 = jnp.ones((16,), jnp.float32)

  def zrow(j, _):
    for k in range(D // 16):
      rows0_v[j, pl.ds(k * 16, 16)] = z16
    return 0
  lax.fori_loop(0, COPY_CH, zrow, 0)

  my_base = s * ROWS_PER_TILE
  zd = []
  for t in range(NCOPY):
    zd.append(pltpu.async_copy(
        rows0_v, m_sh.at[pl.ds(my_base + t * COPY_CH, COPY_CH)], ssem))

  def zdeg(i, _):
    deg_v[pl.ds(i * 16, 16)] = z16
    return 0
  lax.fori_loop(0, NP // 16, zdeg, 0)
  for d in zd:
    d.wait()

  plsc.subcore_barrier()

  def slot(j):
    return (j // BLK_CH) % 2, j % BLK_CH

  def idx_load_start(blk):
    return pltpu.async_copy(idx2_hbm.at[wid, pl.ds(blk * BLK_CH, BLK_CH)],
                            idxb_v.at[blk % 2], isem)

  def gather_start(j):
    b = j % NBUF
    p, r = slot(j)
    return pltpu.async_copy(x_hbm.at[idxb_v.at[p, r, 0]], rows[b], gsem[b])

  def scatter_start(j):
    b = j % NBUF
    p, r = slot(j)
    return pltpu.async_copy(rows[b], m_sh.at[idxb_v.at[p, r, 1]], ssem,
                            add=True)

  def histogram(j):
    p, r = slot(j)
    for k in range(CHUNK // 16):
      idx = idxb_v[p, r, 1, pl.ds(k * 16, 16)]
      plsc.addupdate_scatter(deg_v, [idx], ones16)

  idx_load_start(0).wait()
  ipf = idx_load_start(1)
  ipf_waited = False
  gd = [None] * NBUF
  for j in range(NBUF):
    gd[j] = gather_start(j)

  for j in range(NCHUNK):
    ci = j % BLK_CH
    blk = j // BLK_CH
    if ci == 0 and blk >= 1:
      ipf = idx_load_start(blk + 1) if blk + 1 < NBLK else None
      ipf_waited = False
    if ci == BLK_CH - NBUF and ipf is not None and not ipf_waited:
      ipf.wait()
      ipf_waited = True
    b = j % NBUF
    gd[b].wait()
    sd = scatter_start(j)
    histogram(j)
    sd.wait()
    if j + NBUF < NCHUNK:
      gd[b] = gather_start(j + NBUF)

  pltpu.sync_copy(deg_v, deg_parts_hbm.at[pl.ds(wid * NP, NP)])

  plsc.subcore_barrier()

  ld = [None, None]
  st = [None, None]
  for t in range(NCOPY):
    b = t % 2
    base = my_base + t * COPY_CH
    if st[b] is not None:
      st[b].wait()
    ld[b] = pltpu.async_copy(m_sh.at[pl.ds(base, COPY_CH)], rows[b], gsem[b])
    ld[b].wait()
    st[b] = pltpu.async_copy(rows[b], m_parts_hbm.at[c, pl.ds(base, COPY_CH)],
                             gsem[2 + b])
  st[0].wait()
  st[1].wait()


def _sc_segment_sum(x, idx2):
  mesh = plsc.VectorSubcoreMesh(core_axis_name="c", subcore_axis_name="s")
  fn = functools.partial(
      pl.kernel,
      mesh=mesh,
      compiler_params=pltpu.CompilerParams(
          needs_layout_passes=False, use_tc_tiling_on_sc=False),
      out_type=[
          jax.ShapeDtypeStruct((NUM_CORES, NP, D), jnp.float32),
          jax.ShapeDtypeStruct((NW * NP,), jnp.float32),
      ],
      scratch_types=[
          pltpu.VMEM_SHARED((NP, D), jnp.float32),
          pltpu.VMEM((CHUNK, D), jnp.float32),
          pltpu.VMEM((CHUNK, D), jnp.float32),
          pltpu.VMEM((CHUNK, D), jnp.float32),
          pltpu.VMEM((CHUNK, D), jnp.float32),
          pltpu.VMEM((2, BLK_CH, 2, CHUNK), jnp.int32),
          pltpu.VMEM((NP,), jnp.float32),
          pltpu.SemaphoreType.DMA,
          pltpu.SemaphoreType.DMA,
          pltpu.SemaphoreType.DMA,
          pltpu.SemaphoreType.DMA,
          pltpu.SemaphoreType.DMA,
          pltpu.SemaphoreType.DMA,
      ],
  )(_sc_segment_body)
  return fn(x, idx2)


BLK = 1000


def _tc_body(x_ref, mp_ref, degp_ref, w1_ref, b1_ref, w2_ref, b2_ref, o_ref):
  dn = (((1,), (1,)), ((), ()))
  xn = lax.dot_general(x_ref[...], w1_ref[...], dn,
                       preferred_element_type=jnp.float32) + b1_ref[...]
  m0 = mp_ref[0] + mp_ref[1]
  mm = lax.dot_general(m0, w2_ref[...], dn,
                       preferred_element_type=jnp.float32) + b2_ref[...]
  deg = jnp.sum(degp_ref[...], axis=1)[:, None]
  o_ref[...] = xn + mm / deg


def _tc_combine(x, m_parts, deg_parts, W1, b1, W2, b2):
  grid = (N // BLK,)
  return pl.pallas_call(
      _tc_body,
      grid=grid,
      in_specs=[
          pl.BlockSpec((BLK, D), lambda i: (i, 0)),
          pl.BlockSpec((NUM_CORES, BLK, D), lambda i: (0, i, 0)),
          pl.BlockSpec((BLK, NW), lambda i: (i, 0)),
          pl.BlockSpec((D, D), lambda i: (0, 0)),
          pl.BlockSpec((1, D), lambda i: (0, 0)),
          pl.BlockSpec((D, D), lambda i: (0, 0)),
          pl.BlockSpec((1, D), lambda i: (0, 0)),
      ],
      out_specs=pl.BlockSpec((BLK, D), lambda i: (i, 0)),
      out_shape=jax.ShapeDtypeStruct((N, D), jnp.float32),
  )(x, m_parts, deg_parts, W1, b1, W2, b2)


@jax.jit
def kernel(x, edge_index, W1, b1, W2, b2):
  dst = edge_index[0]
  src = edge_index[1]
  pad = EP - EDGES_PER_W
  srcg = jnp.pad(src.reshape(NW, EDGES_PER_W), ((0, 0), (0, pad)),
                 constant_values=0).reshape(NW, NCHUNK, CHUNK)
  dstg = jnp.pad(dst.reshape(NW, EDGES_PER_W), ((0, 0), (0, pad)),
                 constant_values=N).reshape(NW, NCHUNK, CHUNK)
  idx2 = jnp.stack([srcg, dstg], axis=2)
  m_parts, deg_parts = _sc_segment_sum(x, idx2)
  deg_t = deg_parts.reshape(NW, NP)[:, :N].T
  return _tc_combine(x, m_parts, deg_t,
                     W1, b1.reshape(1, D), W2, b2.reshape(1, D))
